# Initial kernel scaffold; baseline (speedup 1.0000x reference)
import jax
import jax.numpy as jnp
from jax.experimental import pallas as pl


def kernel(feat, score):
    raise NotImplementedError("write your pallas kernel here")


# speedup vs baseline: 2.0784x; 2.0784x over previous
import functools

import jax
import jax.numpy as jnp
from jax import lax
from jax.experimental import pallas as pl
from jax.experimental.pallas import tpu as pltpu
from jax.experimental.pallas import tpu_sc as plsc

B, N, F = 64, 8192, 128
L = 16
NT = N // (L * L)
K_HARD, K_CONF = 10, 5
PAD = 16
ROWS_PER_TILE = 2


def _key_hard(s):
    return jnp.abs(s - 0.5)


def _key_id(s):
    return s


def _phase1(score_v, iota, sh_val, sh_u, sn_val, sn_u, sa_val, sa_u):
    def body(t, _):
        base = t * (L * L)
        run_h = jnp.full((L,), jnp.inf, jnp.float32)
        run_n = jnp.full((L,), jnp.inf, jnp.float32)
        run_a = jnp.full((L,), -jnp.inf, jnp.float32)
        run_hu = jnp.zeros((L,), jnp.int32)
        run_nu = jnp.zeros((L,), jnp.int32)
        run_au = jnp.zeros((L,), jnp.int32)
        for u in range(L):
            s = score_v[pl.ds(base + u * L, L)]
            a = _key_hard(s)
            mh = a < run_h
            run_h = jnp.where(mh, a, run_h)
            run_hu = jnp.where(mh, u, run_hu)
            mn = s < run_n
            run_n = jnp.where(mn, s, run_n)
            run_nu = jnp.where(mn, u, run_nu)
            ma = s > run_a
            run_a = jnp.where(ma, s, run_a)
            run_au = jnp.where(ma, u, run_au)
        sh_val[t] = run_h
        sh_u[t] = run_hu
        sn_val[t] = run_n
        sn_u[t] = run_nu
        sa_val[t] = run_a
        sa_u[t] = run_au
        return 0

    lax.fori_loop(0, NT, body, 0)


def _row_topk(score_v, sval, su, iota, is_min, key_fn, k):
    sentinel = jnp.int32(1 << 30)
    bad = jnp.float32(jnp.inf if is_min else -jnp.inf)
    sel_vec = jnp.zeros((L,), jnp.int32)
    sel_scalars = []
    for p in range(k):
        def body(t, carry):
            rv, rt, ru = carry
            v = sval[t]
            m = (v < rv) if is_min else (v > rv)
            return (jnp.where(m, v, rv),
                    jnp.where(m, t, rt),
                    jnp.where(m, su[t], ru))

        rv, rt, ru = lax.fori_loop(
            0, NT, body,
            (jnp.full((L,), bad, jnp.float32),
             jnp.zeros((L,), jnp.int32), jnp.zeros((L,), jnp.int32)))
        mval = jnp.min(rv) if is_min else jnp.max(rv)
        cand = jnp.where(rv == mval, rt * (L * L) + ru * L + iota, sentinel)
        g = jnp.min(cand)
        sel_vec = jnp.where(iota == p, g, sel_vec)
        sel_scalars.append(g)
        t_sel = lax.shift_right_logical(g, 8)
        l_sel = lax.bitwise_and(g, L - 1)
        bidx = t_sel * (L * L) + iota * L + l_sel
        key = key_fn(plsc.load_gather(score_v, [bidx]))
        excl = bidx == sel_scalars[0]
        for q in sel_scalars[1:]:
            excl = jnp.logical_or(excl, bidx == q)
        keym = jnp.where(excl, bad, key)
        vnew = jnp.min(keym) if is_min else jnp.max(keym)
        ufirst = jnp.min(jnp.where(keym == vnew, iota, jnp.int32(L)))
        lanem = iota == l_sel
        sval[t_sel] = jnp.where(lanem, vnew, sval[t_sel])
        su[t_sel] = jnp.where(lanem, ufirst, su[t_sel])
    return sel_vec


def _body(feat_hbm, score_hbm,
          o_fn, o_sn, o_in, o_fa, o_sa, o_ia, o_fh, o_sh, o_ih,
          score_v, sh_val, sh_u, sn_val, sn_u, sa_val, sa_u,
          idx_v, gidx_v, vals_v, rows_v, sem):
    cid = lax.axis_index("c")
    sid = lax.axis_index("s")
    wid = sid * 2 + cid
    iota = lax.iota(jnp.int32, L)
    for r in range(ROWS_PER_TILE):
        b = wid * ROWS_PER_TILE + r
        pltpu.sync_copy(score_hbm.at[b], score_v)
        _phase1(score_v, iota, sh_val, sh_u, sn_val, sn_u, sa_val, sa_u)
        specs = (
            (True, _key_id, K_CONF, sn_val, sn_u, o_fn, o_sn, o_in),
            (False, _key_id, K_CONF, sa_val, sa_u, o_fa, o_sa, o_ia),
            (True, _key_hard, K_HARD, sh_val, sh_u, o_fh, o_sh, o_ih),
        )
        for is_min, key_fn, k, sval, su, o_f, o_s, o_i in specs:
            sel = _row_topk(score_v, sval, su, iota, is_min, key_fn, k)
            idx_v[...] = sel
            vals_v[...] = plsc.load_gather(score_v, [sel])
            gidx_v[...] = sel + b * N
            pltpu.async_copy(feat_hbm.at[gidx_v], rows_v, sem).wait()
            pltpu.sync_copy(rows_v, o_f.at[b])
            pltpu.sync_copy(vals_v, o_s.at[b])
            pltpu.sync_copy(idx_v, o_i.at[b])


_mesh = plsc.VectorSubcoreMesh(core_axis_name="c", subcore_axis_name="s")

_sc_call = pl.kernel(
    _body,
    out_type=[
        jax.ShapeDtypeStruct((B, PAD, F), jnp.float32),
        jax.ShapeDtypeStruct((B, PAD), jnp.float32),
        jax.ShapeDtypeStruct((B, PAD), jnp.int32),
        jax.ShapeDtypeStruct((B, PAD, F), jnp.float32),
        jax.ShapeDtypeStruct((B, PAD), jnp.float32),
        jax.ShapeDtypeStruct((B, PAD), jnp.int32),
        jax.ShapeDtypeStruct((B, PAD, F), jnp.float32),
        jax.ShapeDtypeStruct((B, PAD), jnp.float32),
        jax.ShapeDtypeStruct((B, PAD), jnp.int32),
    ],
    mesh=_mesh,
    compiler_params=pltpu.CompilerParams(needs_layout_passes=False),
    scratch_types=[
        pltpu.VMEM((N,), jnp.float32),
        pltpu.VMEM((NT, L), jnp.float32),
        pltpu.VMEM((NT, L), jnp.int32),
        pltpu.VMEM((NT, L), jnp.float32),
        pltpu.VMEM((NT, L), jnp.int32),
        pltpu.VMEM((NT, L), jnp.float32),
        pltpu.VMEM((NT, L), jnp.int32),
        pltpu.VMEM((L,), jnp.int32),
        pltpu.VMEM((L,), jnp.int32),
        pltpu.VMEM((L,), jnp.float32),
        pltpu.VMEM((L, F), jnp.float32),
        pltpu.SemaphoreType.DMA,
    ],
)


@jax.jit
def kernel(feat, score):
    feat_flat = feat.reshape(B * N, F)
    fn, sn, i_n, fa, sa, i_a, fh, sh, i_h = _sc_call(feat_flat, score)
    return (fn[:, :K_CONF], sn[:, :K_CONF], i_n[:, :K_CONF],
            fa[:, :K_CONF], sa[:, :K_CONF], i_a[:, :K_CONF],
            fh[:, :K_HARD], sh[:, :K_HARD], i_h[:, :K_HARD])


# speedup vs baseline: 2.1571x; 1.0379x over previous
import functools

import jax
import jax.numpy as jnp
from jax import lax
from jax.experimental import pallas as pl
from jax.experimental.pallas import tpu as pltpu
from jax.experimental.pallas import tpu_sc as plsc

B, N, F = 64, 8192, 128
L = 16
NT = N // (L * L)
K_HARD, K_CONF = 10, 5
PAD = 16
ROWS_PER_TILE = 2


def _key_hard(s):
    return jnp.abs(s - 0.5)


def _key_id(s):
    return s


def _phase1(score_v, iota, sh_val, sh_u, sn_val, sn_u, sa_val, sa_u):
    def body(t, _):
        base = t * (L * L)
        run_h = jnp.full((L,), jnp.inf, jnp.float32)
        run_n = jnp.full((L,), jnp.inf, jnp.float32)
        run_a = jnp.full((L,), -jnp.inf, jnp.float32)
        run_hu = jnp.zeros((L,), jnp.int32)
        run_nu = jnp.zeros((L,), jnp.int32)
        run_au = jnp.zeros((L,), jnp.int32)
        for u in range(L):
            s = score_v[pl.ds(base + u * L, L)]
            a = _key_hard(s)
            mh = a < run_h
            run_h = jnp.where(mh, a, run_h)
            run_hu = jnp.where(mh, u, run_hu)
            mn = s < run_n
            run_n = jnp.where(mn, s, run_n)
            run_nu = jnp.where(mn, u, run_nu)
            ma = s > run_a
            run_a = jnp.where(ma, s, run_a)
            run_au = jnp.where(ma, u, run_au)
        sh_val[t] = run_h
        sh_u[t] = run_hu
        sn_val[t] = run_n
        sn_u[t] = run_nu
        sa_val[t] = run_a
        sa_u[t] = run_au
        return 0

    lax.fori_loop(0, NT, body, 0)


def _row_topk(score_v, sval, su, iota, is_min, key_fn, k):
    sentinel = jnp.int32(1 << 30)
    bad = jnp.float32(jnp.inf if is_min else -jnp.inf)
    sel_vec = jnp.zeros((L,), jnp.int32)
    sel_scalars = []
    for p in range(k):
        def body(t, carry):
            rv, rt, ru = carry
            v = sval[t]
            m = (v < rv) if is_min else (v > rv)
            return (jnp.where(m, v, rv),
                    jnp.where(m, t, rt),
                    jnp.where(m, su[t], ru))

        rv, rt, ru = lax.fori_loop(
            0, NT, body,
            (jnp.full((L,), bad, jnp.float32),
             jnp.zeros((L,), jnp.int32), jnp.zeros((L,), jnp.int32)))
        mval = jnp.min(rv) if is_min else jnp.max(rv)
        cand = jnp.where(rv == mval, rt * (L * L) + ru * L + iota, sentinel)
        g = jnp.min(cand)
        sel_vec = jnp.where(iota == p, g, sel_vec)
        sel_scalars.append(g)
        t_sel = lax.shift_right_logical(g, 8)
        l_sel = lax.bitwise_and(g, L - 1)
        bidx = t_sel * (L * L) + iota * L + l_sel
        key = key_fn(plsc.load_gather(score_v, [bidx]))
        excl = bidx == sel_scalars[0]
        for q in sel_scalars[1:]:
            excl = jnp.logical_or(excl, bidx == q)
        keym = jnp.where(excl, bad, key)
        vnew = jnp.min(keym) if is_min else jnp.max(keym)
        ufirst = jnp.min(jnp.where(keym == vnew, iota, jnp.int32(L)))
        lanem = iota == l_sel
        sval[t_sel] = jnp.where(lanem, vnew, sval[t_sel])
        su[t_sel] = jnp.where(lanem, ufirst, su[t_sel])
    return sel_vec


def _body(feat_hbm, score_hbm,
          o_fn, o_sn, o_in, o_fa, o_sa, o_ia, o_fh, o_sh, o_ih,
          score_v, sh_val, sh_u, sn_val, sn_u, sa_val, sa_u,
          idx_v, gidx_v, vals_v, rows_v, sem):
    cid = lax.axis_index("c")
    sid = lax.axis_index("s")
    wid = sid * 2 + cid
    iota = lax.iota(jnp.int32, L)
    for r in range(ROWS_PER_TILE):
        b = wid * ROWS_PER_TILE + r
        pltpu.sync_copy(score_hbm.at[b], score_v)
        _phase1(score_v, iota, sh_val, sh_u, sn_val, sn_u, sa_val, sa_u)
        specs = (
            (True, _key_id, K_CONF, sn_val, sn_u, 0, o_sn, o_in),
            (False, _key_id, K_CONF, sa_val, sa_u, 1, o_sa, o_ia),
            (True, _key_hard, K_HARD, sh_val, sh_u, 2, o_sh, o_ih),
        )
        for is_min, key_fn, k, sval, su, slot, o_s, o_i in specs:
            sel = _row_topk(score_v, sval, su, iota, is_min, key_fn, k)
            idx_v[...] = sel
            vals_v[...] = plsc.load_gather(score_v, [sel])
            gidx_v[pl.ds(slot * L, L)] = sel + b * N
            pltpu.sync_copy(vals_v, o_s.at[b])
            pltpu.sync_copy(idx_v, o_i.at[b])
        pltpu.async_copy(feat_hbm.at[gidx_v], rows_v, sem).wait()
        pltpu.sync_copy(rows_v.at[pl.ds(0, K_CONF)], o_fn.at[b])
        pltpu.sync_copy(rows_v.at[pl.ds(L, K_CONF)], o_fa.at[b])
        pltpu.sync_copy(rows_v.at[pl.ds(2 * L, K_HARD)], o_fh.at[b])


_mesh = plsc.VectorSubcoreMesh(core_axis_name="c", subcore_axis_name="s")

_sc_call = pl.kernel(
    _body,
    out_type=[
        jax.ShapeDtypeStruct((B, K_CONF, F), jnp.float32),
        jax.ShapeDtypeStruct((B, PAD), jnp.float32),
        jax.ShapeDtypeStruct((B, PAD), jnp.int32),
        jax.ShapeDtypeStruct((B, K_CONF, F), jnp.float32),
        jax.ShapeDtypeStruct((B, PAD), jnp.float32),
        jax.ShapeDtypeStruct((B, PAD), jnp.int32),
        jax.ShapeDtypeStruct((B, K_HARD, F), jnp.float32),
        jax.ShapeDtypeStruct((B, PAD), jnp.float32),
        jax.ShapeDtypeStruct((B, PAD), jnp.int32),
    ],
    mesh=_mesh,
    compiler_params=pltpu.CompilerParams(needs_layout_passes=False),
    scratch_types=[
        pltpu.VMEM((N,), jnp.float32),
        pltpu.VMEM((NT, L), jnp.float32),
        pltpu.VMEM((NT, L), jnp.int32),
        pltpu.VMEM((NT, L), jnp.float32),
        pltpu.VMEM((NT, L), jnp.int32),
        pltpu.VMEM((NT, L), jnp.float32),
        pltpu.VMEM((NT, L), jnp.int32),
        pltpu.VMEM((L,), jnp.int32),
        pltpu.VMEM((3 * L,), jnp.int32),
        pltpu.VMEM((L,), jnp.float32),
        pltpu.VMEM((3 * L, F), jnp.float32),
        pltpu.SemaphoreType.DMA,
    ],
)


@jax.jit
def kernel(feat, score):
    feat_flat = feat.reshape(B * N, F)
    fn, sn, i_n, fa, sa, i_a, fh, sh, i_h = _sc_call(feat_flat, score)
    return (fn, sn[:, :K_CONF], i_n[:, :K_CONF],
            fa, sa[:, :K_CONF], i_a[:, :K_CONF],
            fh, sh[:, :K_HARD], i_h[:, :K_HARD])
